# Initial kernel scaffold; baseline (speedup 1.0000x reference)
import jax
import jax.numpy as jnp
from jax.experimental import pallas as pl


def kernel(a1, w1, w2, topk_weights, topk_ids):
    raise NotImplementedError("write your pallas kernel here")


# speedup vs baseline: 2.6995x; 2.6995x over previous
import functools

import jax
import jax.numpy as jnp
from jax import lax
from jax.experimental import pallas as pl
from jax.experimental.pallas import tpu as pltpu
from jax.experimental.pallas import tpu_sc as plsc

M, DM, DFF, E, TOPK = 2048, 2048, 2048, 8, 2
B = 256
MAXNB = (TOPK * M) // B + E
P = MAXNB * B
NW = 32
TPW = (TOPK * M) // NW
CH = 32


def _meta_body(ids_ref, dest_ref, be_ref):
    ids = ids_ref[...]
    col = lax.broadcasted_iota(jnp.int32, (M, M), 1)
    row = lax.broadcasted_iota(jnp.int32, (M, M), 0)
    tri = (row <= col).astype(jnp.float32)
    occs, cexcls, counts = [], [], []
    for e in range(E):
        occ = (ids == e).astype(jnp.float32)
        c = lax.dot_general(occ, tri, (((1,), (0,)), ((), ())),
                            preferred_element_type=jnp.float32)
        c0_tot = c[0:1, M - 1:M]
        c = c + jnp.concatenate(
            [jnp.zeros((1, M), jnp.float32),
             jnp.broadcast_to(c0_tot, (1, M))], axis=0)
        occs.append(occ)
        cexcls.append(c - occ)
        counts.append(c[1:2, M - 1:M])
    dest = jnp.zeros((TOPK, M), jnp.float32)
    run = jnp.zeros((1, 1), jnp.float32)
    cumnb = []
    for e in range(E):
        dest = dest + occs[e] * (run * B + cexcls[e])
        nb = jnp.floor((counts[e] + (B - 1)) / B)
        run = run + nb
        cumnb.append(run)
    dest_ref[...] = dest.astype(jnp.int32)
    bidx = lax.broadcasted_iota(jnp.int32, (1, MAXNB), 1).astype(jnp.float32)
    be = jnp.zeros((1, MAXNB), jnp.float32)
    for e in range(E):
        be = be + (bidx >= jnp.broadcast_to(cumnb[e], (1, MAXNB))).astype(
            jnp.float32)
    be_ref[...] = be.astype(jnp.int32)


_meta = pl.pallas_call(
    _meta_body,
    out_shape=(jax.ShapeDtypeStruct((TOPK, M), jnp.int32),
               jax.ShapeDtypeStruct((1, MAXNB), jnp.int32)),
)


@functools.cache
def _sc_kernels():
    mesh = plsc.VectorSubcoreMesh(core_axis_name="c", subcore_axis_name="s")

    @functools.partial(
        pl.kernel,
        mesh=mesh,
        out_type=jax.ShapeDtypeStruct((P, DM), jnp.float32),
        scratch_types=[
            pltpu.VMEM((CH,), jnp.int32),
            pltpu.VMEM((CH, DM), jnp.float32),
            pltpu.SemaphoreType.DMA,
        ],
    )
    def dispatch(a1_hbm, destf_hbm, xp_hbm, idx_v, rows_v, sem):
        wid = lax.axis_index("s") * 2 + lax.axis_index("c")
        k = wid // (NW // TOPK)
        t0 = (wid % (NW // TOPK)) * TPW
        for ch in range(TPW // CH):
            base = t0 + ch * CH
            pltpu.sync_copy(destf_hbm.at[pl.ds(k * M + base, CH)], idx_v)
            pltpu.sync_copy(a1_hbm.at[pl.ds(base, CH), :], rows_v)
            pltpu.async_copy(rows_v, xp_hbm.at[idx_v], sem).wait()

    @functools.partial(
        pl.kernel,
        mesh=mesh,
        out_type=jax.ShapeDtypeStruct((TOPK * M, DM), jnp.float32),
        scratch_types=[
            pltpu.VMEM((CH,), jnp.int32),
            pltpu.VMEM((CH, DM), jnp.float32),
            pltpu.SemaphoreType.DMA,
        ],
    )
    def gather2(y2_hbm, destf_hbm, g_hbm, idx_v, rows_v, sem):
        wid = lax.axis_index("s") * 2 + lax.axis_index("c")
        k = wid // (NW // TOPK)
        t0 = (wid % (NW // TOPK)) * TPW
        for ch in range(TPW // CH):
            base = k * M + t0 + ch * CH
            pltpu.sync_copy(destf_hbm.at[pl.ds(base, CH)], idx_v)
            pltpu.async_copy(y2_hbm.at[idx_v], rows_v, sem).wait()
            pltpu.sync_copy(rows_v, g_hbm.at[pl.ds(base, CH), :])

    return dispatch, gather2


def _mm1_body(be_ref, xp_ref, w1_ref, act_ref):
    b = pl.program_id(0)

    @pl.when(be_ref[b] < E)
    def _():
        x = xp_ref[...].astype(jnp.bfloat16)
        h = lax.dot_general(x, w1_ref[0], (((1,), (1,)), ((), ())),
                            preferred_element_type=jnp.float32)
        gate = h[:, :DFF]
        up = h[:, DFF:]
        act_ref[...] = (gate * jax.nn.sigmoid(gate) * up).astype(jnp.bfloat16)


def _mm1_gridspec():
    return pltpu.PrefetchScalarGridSpec(
        num_scalar_prefetch=1,
        grid=(MAXNB,),
        in_specs=[
            pl.BlockSpec((B, DM), lambda b, be: (b, 0)),
            pl.BlockSpec((1, 2 * DFF, DM),
                         lambda b, be: (jnp.minimum(be[b], E - 1), 0, 0)),
        ],
        out_specs=pl.BlockSpec((B, DFF), lambda b, be: (b, 0)),
    )


_mm1 = pl.pallas_call(
    _mm1_body,
    grid_spec=_mm1_gridspec(),
    out_shape=jax.ShapeDtypeStruct((P, DFF), jnp.bfloat16),
)


def _mm2_body(be_ref, act_ref, w2_ref, y2_ref):
    b = pl.program_id(0)

    @pl.when(be_ref[b] < E)
    def _():
        y2_ref[...] = lax.dot_general(
            act_ref[...], w2_ref[0], (((1,), (1,)), ((), ())),
            preferred_element_type=jnp.float32)


def _mm2_gridspec():
    return pltpu.PrefetchScalarGridSpec(
        num_scalar_prefetch=1,
        grid=(MAXNB,),
        in_specs=[
            pl.BlockSpec((B, DFF), lambda b, be: (b, 0)),
            pl.BlockSpec((1, DM, DFF),
                         lambda b, be: (jnp.minimum(be[b], E - 1), 0, 0)),
        ],
        out_specs=pl.BlockSpec((B, DM), lambda b, be: (b, 0)),
    )


_mm2 = pl.pallas_call(
    _mm2_body,
    grid_spec=_mm2_gridspec(),
    out_shape=jax.ShapeDtypeStruct((P, DM), jnp.float32),
)


def _comb_body(g_ref, tw_ref, out_ref):
    g = g_ref[...]
    tw = tw_ref[...]
    out_ref[...] = g[0] * tw[:, 0:1] + g[1] * tw[:, 1:2]


_BM = 256
_comb = pl.pallas_call(
    _comb_body,
    grid=(M // _BM,),
    in_specs=[
        pl.BlockSpec((TOPK, _BM, DM), lambda m: (0, m, 0)),
        pl.BlockSpec((_BM, TOPK), lambda m: (m, 0)),
    ],
    out_specs=pl.BlockSpec((_BM, DM), lambda m: (m, 0)),
    out_shape=jax.ShapeDtypeStruct((M, DM), jnp.float32),
)


def kernel(a1, w1, w2, topk_weights, topk_ids):
    w1b = w1.astype(jnp.bfloat16)
    w2b = w2.astype(jnp.bfloat16)
    dispatch, gather2 = _sc_kernels()
    dest, be = _meta(topk_ids.T)
    destf = dest.reshape(TOPK * M)
    be_arr = be.reshape(MAXNB)
    xp = dispatch(a1, destf)
    act = _mm1(be_arr, xp, w1b)
    y2 = _mm2(be_arr, act, w2b)
    g = gather2(y2, destf)
    return _comb(g.reshape(TOPK, M, DM), topk_weights)
